# baseline (speedup 1.0000x reference)
import functools

import jax
import jax.numpy as jnp
from jax.experimental import pallas as pl

_BM_A = 400
_BM_B = 1000
_EPS = 1e-5


def _mm_stats_body(inp_bf_ref, adj_ref, inrows_ref, out_ref, stats_ref):
    i = pl.program_id(0)

    @pl.when(i == 0)
    def _init():
        stats_ref[...] = jnp.zeros_like(stats_ref)

    a = adj_ref[...].astype(jnp.bfloat16)
    o = jnp.dot(a, inp_bf_ref[...], preferred_element_type=jnp.float32)
    out_ref[...] = o
    xin = inrows_ref[...]
    stats_ref[0:1, :] = stats_ref[0:1, :] + jnp.sum(xin, axis=0, keepdims=True)
    stats_ref[1:2, :] = stats_ref[1:2, :] + jnp.sum(xin * xin, axis=0, keepdims=True)
    stats_ref[2:3, :] = stats_ref[2:3, :] + jnp.sum(o, axis=0, keepdims=True)
    stats_ref[3:4, :] = stats_ref[3:4, :] + jnp.sum(o * o, axis=0, keepdims=True)


def _bn_linear_body(stats_ref, gamma_ref, beta_ref, w1_ref, w2_ref, b_ref,
                    xin_ref, xagg_ref, out_ref, *, n_rows):
    inv_n = 1.0 / n_rows
    mean1 = stats_ref[0:1, :] * inv_n
    var1 = stats_ref[1:2, :] * inv_n - mean1 * mean1
    mean2 = stats_ref[2:3, :] * inv_n
    var2 = stats_ref[3:4, :] * inv_n - mean2 * mean2
    scale1 = gamma_ref[0:1, :] * jax.lax.rsqrt(var1 + _EPS)
    scale2 = gamma_ref[1:2, :] * jax.lax.rsqrt(var2 + _EPS)
    h1 = (xin_ref[...] - mean1) * scale1 + beta_ref[0:1, :]
    h2 = (xagg_ref[...] - mean2) * scale2 + beta_ref[1:2, :]
    dims = (((1,), (1,)), ((), ()))
    d1 = jax.lax.dot_general(h1, w1_ref[...], dims,
                             preferred_element_type=jnp.float32)
    d2 = jax.lax.dot_general(h2, w2_ref[...], dims,
                             preferred_element_type=jnp.float32)
    out_ref[...] = d1 + d2 + b_ref[...]


def kernel(input, adj, gamma, beta, W, b):
    n, d = input.shape
    nb_a = n // _BM_A
    inp_bf = input.astype(jnp.bfloat16)

    agg, stats = pl.pallas_call(
        _mm_stats_body,
        grid=(nb_a,),
        in_specs=[
            pl.BlockSpec((n, d), lambda i: (0, 0)),
            pl.BlockSpec((_BM_A, n), lambda i: (i, 0)),
            pl.BlockSpec((_BM_A, d), lambda i: (i, 0)),
        ],
        out_specs=[
            pl.BlockSpec((_BM_A, d), lambda i: (i, 0)),
            pl.BlockSpec((8, d), lambda i: (0, 0)),
        ],
        out_shape=[
            jax.ShapeDtypeStruct((n, d), jnp.float32),
            jax.ShapeDtypeStruct((8, d), jnp.float32),
        ],
    )(inp_bf, adj, input)

    gamma2 = gamma.reshape(2, d)
    beta2 = beta.reshape(2, d)
    w1 = W[:, :d]
    w2 = W[:, d:]
    b_row = b.reshape(1, d)

    nb_b = n // _BM_B
    out = pl.pallas_call(
        functools.partial(_bn_linear_body, n_rows=float(n)),
        grid=(nb_b,),
        in_specs=[
            pl.BlockSpec((8, d), lambda i: (0, 0)),
            pl.BlockSpec((2, d), lambda i: (0, 0)),
            pl.BlockSpec((2, d), lambda i: (0, 0)),
            pl.BlockSpec((d, d), lambda i: (0, 0)),
            pl.BlockSpec((d, d), lambda i: (0, 0)),
            pl.BlockSpec((1, d), lambda i: (0, 0)),
            pl.BlockSpec((_BM_B, d), lambda i: (i, 0)),
            pl.BlockSpec((_BM_B, d), lambda i: (i, 0)),
        ],
        out_specs=pl.BlockSpec((_BM_B, d), lambda i: (i, 0)),
        out_shape=jax.ShapeDtypeStruct((n, d), jnp.float32),
    )(stats, gamma2, beta2, w1, w2, b_row, input, agg)
    return out


# speedup vs baseline: 1.1043x; 1.1043x over previous
import functools

import jax
import jax.numpy as jnp
from jax.experimental import pallas as pl
from jax.experimental.pallas import tpu as pltpu

_BM_A = 400
_BM_B = 1000
_EPS = 1e-5


def _fused_body(inp_ref, adj_ref, gamma_ref, beta_ref, w1_ref, w2_ref,
                b_ref, out_ref, inp_bf_ref, agg_ref, stats_ref, *,
                nb_a, n_rows):
    i = pl.program_id(0)

    @pl.when(i == 0)
    def _init():
        stats_ref[...] = jnp.zeros_like(stats_ref)
        inp_bf_ref[...] = inp_ref[...].astype(jnp.bfloat16)

    @pl.when(i < nb_a)
    def _phase_a():
        a = adj_ref[...].astype(jnp.bfloat16)
        o = jnp.dot(a, inp_bf_ref[...], preferred_element_type=jnp.float32)
        agg_ref[pl.ds(i * _BM_A, _BM_A), :] = o
        xin = inp_ref[pl.ds(i * _BM_A, _BM_A), :]
        stats_ref[0:1, :] = stats_ref[0:1, :] + jnp.sum(xin, axis=0, keepdims=True)
        stats_ref[1:2, :] = stats_ref[1:2, :] + jnp.sum(xin * xin, axis=0, keepdims=True)
        stats_ref[2:3, :] = stats_ref[2:3, :] + jnp.sum(o, axis=0, keepdims=True)
        stats_ref[3:4, :] = stats_ref[3:4, :] + jnp.sum(o * o, axis=0, keepdims=True)

    @pl.when(i >= nb_a)
    def _phase_b():
        j = i - nb_a
        inv_n = 1.0 / n_rows
        mean1 = stats_ref[0:1, :] * inv_n
        var1 = stats_ref[1:2, :] * inv_n - mean1 * mean1
        mean2 = stats_ref[2:3, :] * inv_n
        var2 = stats_ref[3:4, :] * inv_n - mean2 * mean2
        scale1 = gamma_ref[0:1, :] * jax.lax.rsqrt(var1 + _EPS)
        scale2 = gamma_ref[1:2, :] * jax.lax.rsqrt(var2 + _EPS)
        xin = inp_ref[pl.ds(j * _BM_B, _BM_B), :]
        xagg = agg_ref[pl.ds(j * _BM_B, _BM_B), :]
        h1 = (xin - mean1) * scale1 + beta_ref[0:1, :]
        h2 = (xagg - mean2) * scale2 + beta_ref[1:2, :]
        dims = (((1,), (1,)), ((), ()))
        d1 = jax.lax.dot_general(h1, w1_ref[...], dims,
                                 preferred_element_type=jnp.float32)
        d2 = jax.lax.dot_general(h2, w2_ref[...], dims,
                                 preferred_element_type=jnp.float32)
        out_ref[...] = d1 + d2 + b_ref[...]


def kernel(input, adj, gamma, beta, W, b):
    n, d = input.shape
    nb_a = n // _BM_A
    nb_b = n // _BM_B

    gamma2 = gamma.reshape(2, d)
    beta2 = beta.reshape(2, d)
    w1 = W[:, :d]
    w2 = W[:, d:]
    b_row = b.reshape(1, d)

    last_a = nb_a - 1
    out = pl.pallas_call(
        functools.partial(_fused_body, nb_a=nb_a, n_rows=float(n)),
        grid=(nb_a + nb_b,),
        in_specs=[
            pl.BlockSpec((n, d), lambda i: (0, 0)),
            pl.BlockSpec((_BM_A, n), lambda i: (jnp.minimum(i, last_a), 0)),
            pl.BlockSpec((2, d), lambda i: (0, 0)),
            pl.BlockSpec((2, d), lambda i: (0, 0)),
            pl.BlockSpec((d, d), lambda i: (0, 0)),
            pl.BlockSpec((d, d), lambda i: (0, 0)),
            pl.BlockSpec((1, d), lambda i: (0, 0)),
        ],
        out_specs=pl.BlockSpec(
            (_BM_B, d), lambda i: (jnp.maximum(i - nb_a, 0), 0)),
        out_shape=jax.ShapeDtypeStruct((n, d), jnp.float32),
        scratch_shapes=[
            pltpu.VMEM((n, d), jnp.bfloat16),
            pltpu.VMEM((n, d), jnp.float32),
            pltpu.VMEM((8, d), jnp.float32),
        ],
    )(input, adj, gamma2, beta2, w1, w2, b_row)
    return out


# speedup vs baseline: 1.1085x; 1.0038x over previous
import functools

import jax
import jax.numpy as jnp
from jax.experimental import pallas as pl
from jax.experimental.pallas import tpu as pltpu

_BM_A = 200
_BM_B = 2000
_EPS = 1e-5


def _fused_body(inp_ref, adj_ref, gamma_ref, beta_ref, w1_ref, w2_ref,
                b_ref, out_ref, inp_bf_ref, agg_ref, stats_ref, *,
                nb_a, n_rows):
    i = pl.program_id(0)

    @pl.when(i == 0)
    def _init():
        stats_ref[...] = jnp.zeros_like(stats_ref)
        inp_bf_ref[...] = inp_ref[...].astype(jnp.bfloat16)

    @pl.when(i < nb_a)
    def _phase_a():
        a = adj_ref[...].astype(jnp.bfloat16)
        o = jnp.dot(a, inp_bf_ref[...], preferred_element_type=jnp.float32)
        agg_ref[pl.ds(i * _BM_A, _BM_A), :] = o
        xin = inp_ref[pl.ds(i * _BM_A, _BM_A), :]
        stats_ref[0:1, :] = stats_ref[0:1, :] + jnp.sum(xin, axis=0, keepdims=True)
        stats_ref[1:2, :] = stats_ref[1:2, :] + jnp.sum(xin * xin, axis=0, keepdims=True)
        stats_ref[2:3, :] = stats_ref[2:3, :] + jnp.sum(o, axis=0, keepdims=True)
        stats_ref[3:4, :] = stats_ref[3:4, :] + jnp.sum(o * o, axis=0, keepdims=True)

    @pl.when(i >= nb_a)
    def _phase_b():
        j = i - nb_a
        inv_n = 1.0 / n_rows
        mean1 = stats_ref[0:1, :] * inv_n
        var1 = stats_ref[1:2, :] * inv_n - mean1 * mean1
        mean2 = stats_ref[2:3, :] * inv_n
        var2 = stats_ref[3:4, :] * inv_n - mean2 * mean2
        scale1 = gamma_ref[0:1, :] * jax.lax.rsqrt(var1 + _EPS)
        scale2 = gamma_ref[1:2, :] * jax.lax.rsqrt(var2 + _EPS)
        xin = inp_ref[pl.ds(j * _BM_B, _BM_B), :]
        xagg = agg_ref[pl.ds(j * _BM_B, _BM_B), :]
        h1 = (xin - mean1) * scale1 + beta_ref[0:1, :]
        h2 = (xagg - mean2) * scale2 + beta_ref[1:2, :]
        dims = (((1,), (1,)), ((), ()))
        d1 = jax.lax.dot_general(h1, w1_ref[...], dims,
                                 preferred_element_type=jnp.float32)
        d2 = jax.lax.dot_general(h2, w2_ref[...], dims,
                                 preferred_element_type=jnp.float32)
        out_ref[...] = d1 + d2 + b_ref[...]


def kernel(input, adj, gamma, beta, W, b):
    n, d = input.shape
    nb_a = n // _BM_A
    nb_b = n // _BM_B

    gamma2 = gamma.reshape(2, d)
    beta2 = beta.reshape(2, d)
    w1 = W[:, :d]
    w2 = W[:, d:]
    b_row = b.reshape(1, d)

    last_a = nb_a - 1
    out = pl.pallas_call(
        functools.partial(_fused_body, nb_a=nb_a, n_rows=float(n)),
        grid=(nb_a + nb_b,),
        in_specs=[
            pl.BlockSpec((n, d), lambda i: (0, 0)),
            pl.BlockSpec((_BM_A, n), lambda i: (jnp.minimum(i, last_a), 0)),
            pl.BlockSpec((2, d), lambda i: (0, 0)),
            pl.BlockSpec((2, d), lambda i: (0, 0)),
            pl.BlockSpec((d, d), lambda i: (0, 0)),
            pl.BlockSpec((d, d), lambda i: (0, 0)),
            pl.BlockSpec((1, d), lambda i: (0, 0)),
        ],
        out_specs=pl.BlockSpec(
            (_BM_B, d), lambda i: (jnp.maximum(i - nb_a, 0), 0)),
        out_shape=jax.ShapeDtypeStruct((n, d), jnp.float32),
        scratch_shapes=[
            pltpu.VMEM((n, d), jnp.bfloat16),
            pltpu.VMEM((n, d), jnp.float32),
            pltpu.VMEM((8, d), jnp.float32),
        ],
    )(input, adj, gamma2, beta2, w1, w2, b_row)
    return out
